# baseline (speedup 1.0000x reference)
import jax
import jax.numpy as jnp
from jax import lax
from jax.experimental import pallas as pl
from jax.experimental.pallas import tpu as pltpu
from jax.experimental.pallas import tpu_sc as plsc

NC = 2
NS = 16
L = 16
NT = NC * NS

N = 10000
E = 320000
D = 128
TPE = E // NT
CH = 40
CPT = TPE // CH
AW = 16
SEG = 640
NPAD = 10240
DTILES = 10
DROWS = N // DTILES

_SC_PARAMS = pltpu.CompilerParams(needs_layout_passes=False)
_MESH = plsc.VectorSubcoreMesh(core_axis_name="c", subcore_axis_name="s")


def _mm_body(feat_ref, w_ref, a_ref, x_ref, a2_ref):
    x = jnp.dot(feat_ref[:], w_ref[:], preferred_element_type=jnp.float32)
    x_ref[:] = x
    a2_ref[:] = jnp.dot(x, a_ref[:], preferred_element_type=jnp.float32)


def _final_body(p_ref, b_ref, o_ref):
    o_ref[:] = p_ref[0] + p_ref[1] + b_ref[:]


def _vshift(v, idx):
    dn = lax.GatherDimensionNumbers(offset_dims=(), collapsed_slice_dims=(0,),
                                    start_index_map=(0,))
    return lax.gather(v, idx[:, None], dn, (1,),
                      mode=lax.GatherScatterMode.PROMISE_IN_BOUNDS)


def _phase_a(src_h, dst_h, asrc_h, adst_h, aexp_h, asump_h,
             src_v, dst_v, asrc_v, adst_v, aexp_v, asum_v, blk_v, asum_sh2):
    c = lax.axis_index("c")
    s = lax.axis_index("s")
    tile = c * NS + s
    base = tile * TPE

    pltpu.sync_copy(src_h.at[pl.ds(base, TPE)], src_v)
    pltpu.sync_copy(dst_h.at[pl.ds(base, TPE)], dst_v)
    pltpu.sync_copy(asrc_h, asrc_v)
    pltpu.sync_copy(adst_h, adst_v)

    def zb(i, _):
        asum_v[pl.ds(i * L, L)] = jnp.zeros((L,), jnp.float32)
        return 0
    lax.fori_loop(0, NPAD // L, zb, 0)

    iota = lax.iota(jnp.int32, L)
    im1 = jnp.maximum(iota - 1, 0)
    ip1 = jnp.minimum(iota + 1, L - 1)

    def body(i, _):
        s16 = src_v[pl.ds(i * L, L)]
        d16 = dst_v[pl.ds(i * L, L)]
        av = plsc.load_gather(asrc_v, [s16]) + plsc.load_gather(adst_v, [d16])
        alpha = jnp.where(av >= 0.0, av, av * 0.2)
        aexp16 = jnp.exp(alpha)
        aexp_v[pl.ds(i * L, L)] = aexp16

        keys, vals = plsc.sort_key_val(d16, aexp16)
        seg = plsc.cumsum(vals)
        segm1 = jnp.where(iota == 0, 0.0, _vshift(seg, im1))
        keym1 = jnp.where(iota == 0, -1, _vshift(keys, im1))
        first = keys != keym1
        base16 = plsc.cummax(jnp.where(first, segm1, 0.0))
        tot = seg - base16
        keyp1 = jnp.where(iota == L - 1, -1, _vshift(keys, ip1))
        last = keys != keyp1
        plsc.addupdate_scatter(asum_v, [keys], tot, mask=last)
        return 0
    lax.fori_loop(0, TPE // L, body, 0)

    pltpu.sync_copy(aexp_v, aexp_h.at[pl.ds(base, TPE)])

    pltpu.sync_copy(asum_v, asum_sh2.at[s, 0])
    plsc.subcore_barrier()
    pltpu.sync_copy(asum_sh2.at[:, 0, pl.ds(s * SEG, SEG)], blk_v)

    def rsum(g, _):
        sl = pl.ds(g * L, L)
        acc = blk_v[0, sl]
        for r in range(1, NS):
            acc = acc + blk_v[r, sl]
        blk_v[0, sl] = acc
        return 0
    lax.fori_loop(0, SEG // L, rsum, 0)
    pltpu.sync_copy(blk_v.at[0], asump_h.at[c, pl.ds(s * SEG, SEG)])


def _phase_a2(dst_h, aexp_h, asump_h, coef_h, dst_v, asum_v, part_v, work_v):
    c = lax.axis_index("c")
    s = lax.axis_index("s")
    tile = c * NS + s
    base = tile * TPE

    pltpu.sync_copy(dst_h.at[pl.ds(base, TPE)], dst_v)
    pltpu.sync_copy(asump_h.at[0], asum_v)
    pltpu.sync_copy(asump_h.at[1], part_v)

    def addp(i, _):
        sl = pl.ds(i * L, L)
        asum_v[sl] = asum_v[sl] + part_v[sl]
        return 0
    lax.fori_loop(0, N // L, addp, 0)

    pltpu.sync_copy(aexp_h.at[pl.ds(base, TPE)], work_v)

    def body(i, _):
        sl = pl.ds(i * L, L)
        d16 = dst_v[sl]
        sm = plsc.load_gather(asum_v, [d16])
        work_v[sl] = work_v[sl] / (sm + 1e-16)
        return 0
    lax.fori_loop(0, TPE // L, body, 0)

    pltpu.sync_copy(work_v, coef_h.at[pl.ds(base, TPE)])


def _phase_b(src_h, dst_h, coef_h, x_h, outp_h,
             src_v, dst_v, coef_v, r0, r1, r2, r3, r4, r5, r6, r7, r8, r9,
             g0, g1, g2, g3, g4, g5, g6, g7, g8, g9,
             s0, s1, s2, s3, s4, s5, s6, s7, s8, s9, out_sh):
    c = lax.axis_index("c")
    s = lax.axis_index("s")
    tile = c * NS + s
    base = tile * TPE
    bufs = (r0, r1, r2, r3, r4, r5, r6, r7, r8, r9)
    gsems = (g0, g1, g2, g3, g4, g5, g6, g7, g8, g9)
    ssems = (s0, s1, s2, s3, s4, s5, s6, s7, s8, s9)
    NB = 10
    NCHUNK = TPE // L

    pltpu.sync_copy(src_h.at[pl.ds(base, TPE)], src_v)
    pltpu.sync_copy(dst_h.at[pl.ds(base, TPE)], dst_v)
    pltpu.sync_copy(coef_h.at[pl.ds(base, TPE)], coef_v)

    @pl.when(s < DTILES)
    def _():
        def zrow(j, _):
            for c8 in range(D // L):
                r0[j, pl.ds(c8 * L, L)] = jnp.zeros((L,), jnp.float32)
            return 0
        lax.fori_loop(0, L, zrow, 0)
        for i in range(DROWS // L):
            pltpu.sync_copy(r0, out_sh.at[pl.ds(s * DROWS + i * L, L)])
        if DROWS % L:
            pltpu.sync_copy(r0.at[pl.ds(0, DROWS % L)],
                            out_sh.at[pl.ds(s * DROWS + (DROWS // L) * L, DROWS % L)])

    plsc.subcore_barrier()

    def start_gather(k, buf, sem):
        pltpu.async_copy(x_h.at[src_v[pl.ds(k * L, L)]], buf, sem)

    def wait_gather(k, buf, sem):
        pltpu.make_async_copy(x_h.at[src_v[pl.ds(k * L, L)]], buf, sem).wait()

    def start_scatter(k, buf, sem):
        pltpu.async_copy(buf, out_sh.at[dst_v[pl.ds(k * L, L)]], sem, add=True)

    def wait_scatter(k, buf, sem):
        pltpu.make_async_copy(buf, out_sh.at[dst_v[pl.ds(k * L, L)]], sem).wait()

    def scale(k, buf):
        cs16 = coef_v[pl.ds(k * L, L)]
        for j in range(L):
            csj = _vshift(cs16, jnp.full((L,), j, jnp.int32))
            for c8 in range(D // L):
                sl = pl.ds(c8 * L, L)
                buf[j, sl] = buf[j, sl] * csj

    for b in range(NB):
        start_gather(b, bufs[b], gsems[b])

    def lbody(h, _):
        k0 = NB * h
        for b in range(NB):
            wait_gather(k0 + b, bufs[b], gsems[b])
            scale(k0 + b, bufs[b])
            start_scatter(k0 + b, bufs[b], ssems[b])
        for b in range(NB):
            wait_scatter(k0 + b, bufs[b], ssems[b])
            start_gather(k0 + NB + b, bufs[b], gsems[b])
        return 0
    lax.fori_loop(0, NCHUNK // NB - 1, lbody, 0)

    kt = (NCHUNK // NB - 1) * NB
    for b in range(NB):
        wait_gather(kt + b, bufs[b], gsems[b])
        scale(kt + b, bufs[b])
        start_scatter(kt + b, bufs[b], ssems[b])
    wait_scatter(kt, bufs[0], ssems[0])
    for k in range(kt + NB, NCHUNK):
        b = k % NB
        start_gather(k, bufs[b], gsems[b])
        wait_gather(k, bufs[b], gsems[b])
        scale(k, bufs[b])
        start_scatter(k, bufs[b], ssems[b])
        wait_scatter(k, bufs[b], ssems[b])
    for b in range(1, NB):
        wait_scatter(kt + b, bufs[b], ssems[b])

    plsc.subcore_barrier()

    @pl.when(s < DTILES)
    def _():
        pltpu.sync_copy(out_sh.at[pl.ds(s * DROWS, DROWS)],
                        outp_h.at[c, pl.ds(s * DROWS, DROWS)])


@jax.jit
def _run(in_feat, edge_list, W, att_src, att_dst, bias):
    att_pad = jnp.zeros((D, D), jnp.float32)
    att_pad = att_pad.at[:, 0].set(att_src).at[:, 1].set(att_dst)

    BLK = 1000
    x, a2 = pl.pallas_call(
        _mm_body,
        grid=(N // BLK,),
        in_specs=[
            pl.BlockSpec((BLK, D), lambda i: (i, 0)),
            pl.BlockSpec((D, D), lambda i: (0, 0)),
            pl.BlockSpec((D, D), lambda i: (0, 0)),
        ],
        out_specs=[
            pl.BlockSpec((BLK, D), lambda i: (i, 0)),
            pl.BlockSpec((BLK, D), lambda i: (i, 0)),
        ],
        out_shape=[
            jax.ShapeDtypeStruct((N, D), jnp.float32),
            jax.ShapeDtypeStruct((N, D), jnp.float32),
        ],
    )(in_feat, W, att_pad)

    asrc = a2[:, 0] + 0.0
    adst = a2[:, 1] + 0.0
    src = edge_list[0]
    dst = edge_list[1]

    phase_a = pl.kernel(
        _phase_a,
        out_type=[
            jax.ShapeDtypeStruct((E,), jnp.float32),
            jax.ShapeDtypeStruct((NC, NPAD), jnp.float32),
        ],
        mesh=_MESH,
        compiler_params=_SC_PARAMS,
        scratch_types=[
            pltpu.VMEM((TPE,), jnp.int32),
            pltpu.VMEM((TPE,), jnp.int32),
            pltpu.VMEM((N,), jnp.float32),
            pltpu.VMEM((N,), jnp.float32),
            pltpu.VMEM((TPE,), jnp.float32),
            pltpu.VMEM((NPAD,), jnp.float32),
            pltpu.VMEM((NS, SEG), jnp.float32),
            pltpu.VMEM_SHARED((NS, 1, NPAD), jnp.float32),
        ],
    )
    aexp, asump = phase_a(src, dst, asrc, adst)

    phase_a2 = pl.kernel(
        _phase_a2,
        out_type=jax.ShapeDtypeStruct((E,), jnp.float32),
        mesh=_MESH,
        compiler_params=_SC_PARAMS,
        scratch_types=[
            pltpu.VMEM((TPE,), jnp.int32),
            pltpu.VMEM((NPAD,), jnp.float32),
            pltpu.VMEM((NPAD,), jnp.float32),
            pltpu.VMEM((TPE,), jnp.float32),
        ],
    )
    coef = phase_a2(dst, aexp, asump)

    phase_b = pl.kernel(
        _phase_b,
        out_type=jax.ShapeDtypeStruct((NC, N, D), jnp.float32),
        mesh=_MESH,
        compiler_params=_SC_PARAMS,
        scratch_types=[
            pltpu.VMEM((TPE,), jnp.int32),
            pltpu.VMEM((TPE,), jnp.int32),
            pltpu.VMEM((TPE,), jnp.float32),
            pltpu.VMEM((L, D), jnp.float32),
            pltpu.VMEM((L, D), jnp.float32),
            pltpu.VMEM((L, D), jnp.float32),
            pltpu.VMEM((L, D), jnp.float32),
            pltpu.VMEM((L, D), jnp.float32),
            pltpu.VMEM((L, D), jnp.float32),
            pltpu.VMEM((L, D), jnp.float32),
            pltpu.VMEM((L, D), jnp.float32),
            pltpu.VMEM((L, D), jnp.float32),
            pltpu.VMEM((L, D), jnp.float32),
            pltpu.SemaphoreType.DMA,
            pltpu.SemaphoreType.DMA,
            pltpu.SemaphoreType.DMA,
            pltpu.SemaphoreType.DMA,
            pltpu.SemaphoreType.DMA,
            pltpu.SemaphoreType.DMA,
            pltpu.SemaphoreType.DMA,
            pltpu.SemaphoreType.DMA,
            pltpu.SemaphoreType.DMA,
            pltpu.SemaphoreType.DMA,
            pltpu.SemaphoreType.DMA,
            pltpu.SemaphoreType.DMA,
            pltpu.SemaphoreType.DMA,
            pltpu.SemaphoreType.DMA,
            pltpu.SemaphoreType.DMA,
            pltpu.SemaphoreType.DMA,
            pltpu.SemaphoreType.DMA,
            pltpu.SemaphoreType.DMA,
            pltpu.SemaphoreType.DMA,
            pltpu.SemaphoreType.DMA,
            pltpu.VMEM_SHARED((N, D), jnp.float32),
        ],
    )
    outp = phase_b(src, dst, coef, x)

    out = pl.pallas_call(
        _final_body,
        grid=(N // BLK,),
        in_specs=[
            pl.BlockSpec((NC, BLK, D), lambda i: (0, i, 0)),
            pl.BlockSpec((1, D), lambda i: (0, 0)),
        ],
        out_specs=pl.BlockSpec((BLK, D), lambda i: (i, 0)),
        out_shape=jax.ShapeDtypeStruct((N, D), jnp.float32),
    )(outp, bias.reshape(1, D))
    return out


def kernel(in_feat, edge_list, W, att_src, att_dst, bias):
    return _run(in_feat, edge_list, W, att_src, att_dst, bias)


# speedup vs baseline: 1.0363x; 1.0363x over previous
import jax
import jax.numpy as jnp
from jax import lax
from jax.experimental import pallas as pl
from jax.experimental.pallas import tpu as pltpu
from jax.experimental.pallas import tpu_sc as plsc

NC = 2
NS = 16
L = 16
NT = NC * NS

N = 10000
E = 320000
D = 128
TPE = E // NT
CH = 40
CPT = TPE // CH
AW = 16
SEG = 640
NPAD = 10240
DTILES = 10
DROWS = N // DTILES

_SC_PARAMS = pltpu.CompilerParams(needs_layout_passes=False)
_MESH = plsc.VectorSubcoreMesh(core_axis_name="c", subcore_axis_name="s")


def _mm_body(feat_ref, w_ref, a_ref, x_ref, a2_ref):
    x = jnp.dot(feat_ref[:], w_ref[:], preferred_element_type=jnp.float32)
    x_ref[:] = x
    a2_ref[:] = jnp.dot(x, a_ref[:], preferred_element_type=jnp.float32)


def _final_body(p_ref, b_ref, o_ref):
    o_ref[:] = p_ref[0] + p_ref[1] + b_ref[:]


def _vshift(v, idx):
    dn = lax.GatherDimensionNumbers(offset_dims=(), collapsed_slice_dims=(0,),
                                    start_index_map=(0,))
    return lax.gather(v, idx[:, None], dn, (1,),
                      mode=lax.GatherScatterMode.PROMISE_IN_BOUNDS)


def _phase_a(src_h, dst_h, asrc_h, adst_h, aexp_h, asump_h,
             src_v, dst_v, asrc_v, adst_v, aexp_v, asum_v, blk_v, lsem, asum_sh2):
    c = lax.axis_index("c")
    s = lax.axis_index("s")
    tile = c * NS + s
    base = tile * TPE

    ld0 = pltpu.async_copy(src_h.at[pl.ds(base, TPE)], src_v, lsem)
    ld1 = pltpu.async_copy(dst_h.at[pl.ds(base, TPE)], dst_v, lsem)
    ld2 = pltpu.async_copy(asrc_h, asrc_v, lsem)
    ld3 = pltpu.async_copy(adst_h, adst_v, lsem)

    def zb(i, _):
        asum_v[pl.ds(i * L, L)] = jnp.zeros((L,), jnp.float32)
        return 0
    lax.fori_loop(0, NPAD // L, zb, 0)
    ld0.wait(); ld1.wait(); ld2.wait(); ld3.wait()

    iota = lax.iota(jnp.int32, L)
    im1 = jnp.maximum(iota - 1, 0)
    ip1 = jnp.minimum(iota + 1, L - 1)

    def body(i, _):
        s16 = src_v[pl.ds(i * L, L)]
        d16 = dst_v[pl.ds(i * L, L)]
        av = plsc.load_gather(asrc_v, [s16]) + plsc.load_gather(adst_v, [d16])
        alpha = jnp.where(av >= 0.0, av, av * 0.2)
        aexp16 = jnp.exp(alpha)
        aexp_v[pl.ds(i * L, L)] = aexp16

        keys, vals = plsc.sort_key_val(d16, aexp16)
        seg = plsc.cumsum(vals)
        segm1 = jnp.where(iota == 0, 0.0, _vshift(seg, im1))
        keym1 = jnp.where(iota == 0, -1, _vshift(keys, im1))
        first = keys != keym1
        base16 = plsc.cummax(jnp.where(first, segm1, 0.0))
        tot = seg - base16
        keyp1 = jnp.where(iota == L - 1, -1, _vshift(keys, ip1))
        last = keys != keyp1
        plsc.addupdate_scatter(asum_v, [keys], tot, mask=last)
        return 0
    lax.fori_loop(0, TPE // L, body, 0)

    pltpu.sync_copy(aexp_v, aexp_h.at[pl.ds(base, TPE)])

    pltpu.sync_copy(asum_v, asum_sh2.at[s, 0])
    plsc.subcore_barrier()
    pltpu.sync_copy(asum_sh2.at[:, 0, pl.ds(s * SEG, SEG)], blk_v)

    def rsum(g, _):
        sl = pl.ds(g * L, L)
        acc = blk_v[0, sl]
        for r in range(1, NS):
            acc = acc + blk_v[r, sl]
        blk_v[0, sl] = acc
        return 0
    lax.fori_loop(0, SEG // L, rsum, 0)
    pltpu.sync_copy(blk_v.at[0], asump_h.at[c, pl.ds(s * SEG, SEG)])


def _phase_a2(dst_h, aexp_h, asump_h, coef_h, dst_v, asum_v, part_v, work_v, lsem):
    c = lax.axis_index("c")
    s = lax.axis_index("s")
    tile = c * NS + s
    base = tile * TPE

    ld0 = pltpu.async_copy(dst_h.at[pl.ds(base, TPE)], dst_v, lsem)
    ld1 = pltpu.async_copy(asump_h.at[0], asum_v, lsem)
    ld2 = pltpu.async_copy(asump_h.at[1], part_v, lsem)
    ld3 = pltpu.async_copy(aexp_h.at[pl.ds(base, TPE)], work_v, lsem)
    ld0.wait(); ld1.wait(); ld2.wait(); ld3.wait()

    def addp(i, _):
        sl = pl.ds(i * L, L)
        asum_v[sl] = asum_v[sl] + part_v[sl]
        return 0
    lax.fori_loop(0, N // L, addp, 0)

    def body(i, _):
        sl = pl.ds(i * L, L)
        d16 = dst_v[sl]
        sm = plsc.load_gather(asum_v, [d16])
        work_v[sl] = work_v[sl] / (sm + 1e-16)
        return 0
    lax.fori_loop(0, TPE // L, body, 0)

    pltpu.sync_copy(work_v, coef_h.at[pl.ds(base, TPE)])


def _phase_b(src_h, dst_h, coef_h, x_h, outp_h,
             src_v, dst_v, coef_v, r0, r1, r2, r3, r4, r5, r6, r7,
             g0, g1, g2, g3, g4, g5, g6, g7,
             s0, s1, s2, s3, s4, s5, s6, s7, out_sh):
    c = lax.axis_index("c")
    s = lax.axis_index("s")
    tile = c * NS + s
    base = tile * TPE
    bufs = (r0, r1, r2, r3, r4, r5, r6, r7)
    gsems = (g0, g1, g2, g3, g4, g5, g6, g7)
    ssems = (s0, s1, s2, s3, s4, s5, s6, s7)
    NB = 8
    NCHUNK = TPE // L

    ld0 = pltpu.async_copy(src_h.at[pl.ds(base, TPE)], src_v, g0)
    ld1 = pltpu.async_copy(dst_h.at[pl.ds(base, TPE)], dst_v, g1)
    ld2 = pltpu.async_copy(coef_h.at[pl.ds(base, TPE)], coef_v, g2)

    @pl.when(s < DTILES)
    def _():
        def zrow(j, _):
            for c8 in range(D // L):
                r0[j, pl.ds(c8 * L, L)] = jnp.zeros((L,), jnp.float32)
            return 0
        lax.fori_loop(0, L, zrow, 0)
        for i in range(DROWS // L):
            pltpu.sync_copy(r0, out_sh.at[pl.ds(s * DROWS + i * L, L)])
        if DROWS % L:
            pltpu.sync_copy(r0.at[pl.ds(0, DROWS % L)],
                            out_sh.at[pl.ds(s * DROWS + (DROWS // L) * L, DROWS % L)])

    ld0.wait(); ld1.wait(); ld2.wait()
    plsc.subcore_barrier()

    def start_gather(k, buf, sem):
        pltpu.async_copy(x_h.at[src_v[pl.ds(k * L, L)]], buf, sem)

    def wait_gather(k, buf, sem):
        pltpu.make_async_copy(x_h.at[src_v[pl.ds(k * L, L)]], buf, sem).wait()

    def start_scatter(k, buf, sem):
        pltpu.async_copy(buf, out_sh.at[dst_v[pl.ds(k * L, L)]], sem, add=True)

    def wait_scatter(k, buf, sem):
        pltpu.make_async_copy(buf, out_sh.at[dst_v[pl.ds(k * L, L)]], sem).wait()

    def scale(k, buf):
        cs16 = coef_v[pl.ds(k * L, L)]
        for j in range(L):
            csj = _vshift(cs16, jnp.full((L,), j, jnp.int32))
            for c8 in range(D // L):
                sl = pl.ds(c8 * L, L)
                buf[j, sl] = buf[j, sl] * csj

    for b in range(NB):
        start_gather(b, bufs[b], gsems[b])

    def lbody(h, _):
        k0 = NB * h
        for b in range(NB):
            wait_gather(k0 + b, bufs[b], gsems[b])
            scale(k0 + b, bufs[b])
            start_scatter(k0 + b, bufs[b], ssems[b])
        for b in range(NB):
            wait_scatter(k0 + b, bufs[b], ssems[b])
            start_gather(k0 + NB + b, bufs[b], gsems[b])
        return 0
    lax.fori_loop(0, NCHUNK // NB - 1, lbody, 0)

    kt = (NCHUNK // NB - 1) * NB
    for b in range(NB):
        wait_gather(kt + b, bufs[b], gsems[b])
        scale(kt + b, bufs[b])
        start_scatter(kt + b, bufs[b], ssems[b])
    wait_scatter(kt, bufs[0], ssems[0])
    for k in range(kt + NB, NCHUNK):
        b = k % NB
        start_gather(k, bufs[b], gsems[b])
        wait_gather(k, bufs[b], gsems[b])
        scale(k, bufs[b])
        start_scatter(k, bufs[b], ssems[b])
        wait_scatter(k, bufs[b], ssems[b])
    for b in range(1, NB):
        wait_scatter(kt + b, bufs[b], ssems[b])

    plsc.subcore_barrier()

    @pl.when(s < DTILES)
    def _():
        pltpu.sync_copy(out_sh.at[pl.ds(s * DROWS, DROWS)],
                        outp_h.at[c, pl.ds(s * DROWS, DROWS)])


@jax.jit
def _run(in_feat, edge_list, W, att_src, att_dst, bias):
    att_pad = jnp.zeros((D, D), jnp.float32)
    att_pad = att_pad.at[:, 0].set(att_src).at[:, 1].set(att_dst)

    BLK = 1000
    x, a2 = pl.pallas_call(
        _mm_body,
        grid=(N // BLK,),
        in_specs=[
            pl.BlockSpec((BLK, D), lambda i: (i, 0)),
            pl.BlockSpec((D, D), lambda i: (0, 0)),
            pl.BlockSpec((D, D), lambda i: (0, 0)),
        ],
        out_specs=[
            pl.BlockSpec((BLK, D), lambda i: (i, 0)),
            pl.BlockSpec((BLK, D), lambda i: (i, 0)),
        ],
        out_shape=[
            jax.ShapeDtypeStruct((N, D), jnp.float32),
            jax.ShapeDtypeStruct((N, D), jnp.float32),
        ],
    )(in_feat, W, att_pad)

    asrc = a2[:, 0] + 0.0
    adst = a2[:, 1] + 0.0
    src = edge_list[0]
    dst = edge_list[1]

    phase_a = pl.kernel(
        _phase_a,
        out_type=[
            jax.ShapeDtypeStruct((E,), jnp.float32),
            jax.ShapeDtypeStruct((NC, NPAD), jnp.float32),
        ],
        mesh=_MESH,
        compiler_params=_SC_PARAMS,
        scratch_types=[
            pltpu.VMEM((TPE,), jnp.int32),
            pltpu.VMEM((TPE,), jnp.int32),
            pltpu.VMEM((N,), jnp.float32),
            pltpu.VMEM((N,), jnp.float32),
            pltpu.VMEM((TPE,), jnp.float32),
            pltpu.VMEM((NPAD,), jnp.float32),
            pltpu.VMEM((NS, SEG), jnp.float32),
            pltpu.SemaphoreType.DMA,
            pltpu.VMEM_SHARED((NS, 1, NPAD), jnp.float32),
        ],
    )
    aexp, asump = phase_a(src, dst, asrc, adst)

    phase_a2 = pl.kernel(
        _phase_a2,
        out_type=jax.ShapeDtypeStruct((E,), jnp.float32),
        mesh=_MESH,
        compiler_params=_SC_PARAMS,
        scratch_types=[
            pltpu.VMEM((TPE,), jnp.int32),
            pltpu.VMEM((NPAD,), jnp.float32),
            pltpu.VMEM((NPAD,), jnp.float32),
            pltpu.VMEM((TPE,), jnp.float32),
            pltpu.SemaphoreType.DMA,
        ],
    )
    coef = phase_a2(dst, aexp, asump)

    phase_b = pl.kernel(
        _phase_b,
        out_type=jax.ShapeDtypeStruct((NC, N, D), jnp.float32),
        mesh=_MESH,
        compiler_params=_SC_PARAMS,
        scratch_types=[
            pltpu.VMEM((TPE,), jnp.int32),
            pltpu.VMEM((TPE,), jnp.int32),
            pltpu.VMEM((TPE,), jnp.float32),
            pltpu.VMEM((L, D), jnp.float32),
            pltpu.VMEM((L, D), jnp.float32),
            pltpu.VMEM((L, D), jnp.float32),
            pltpu.VMEM((L, D), jnp.float32),
            pltpu.VMEM((L, D), jnp.float32),
            pltpu.VMEM((L, D), jnp.float32),
            pltpu.VMEM((L, D), jnp.float32),
            pltpu.VMEM((L, D), jnp.float32),
            pltpu.SemaphoreType.DMA,
            pltpu.SemaphoreType.DMA,
            pltpu.SemaphoreType.DMA,
            pltpu.SemaphoreType.DMA,
            pltpu.SemaphoreType.DMA,
            pltpu.SemaphoreType.DMA,
            pltpu.SemaphoreType.DMA,
            pltpu.SemaphoreType.DMA,
            pltpu.SemaphoreType.DMA,
            pltpu.SemaphoreType.DMA,
            pltpu.SemaphoreType.DMA,
            pltpu.SemaphoreType.DMA,
            pltpu.SemaphoreType.DMA,
            pltpu.SemaphoreType.DMA,
            pltpu.SemaphoreType.DMA,
            pltpu.SemaphoreType.DMA,
            pltpu.VMEM_SHARED((N, D), jnp.float32),
        ],
    )
    outp = phase_b(src, dst, coef, x)

    out = pl.pallas_call(
        _final_body,
        grid=(N // BLK,),
        in_specs=[
            pl.BlockSpec((NC, BLK, D), lambda i: (0, i, 0)),
            pl.BlockSpec((1, D), lambda i: (0, 0)),
        ],
        out_specs=pl.BlockSpec((BLK, D), lambda i: (i, 0)),
        out_shape=jax.ShapeDtypeStruct((N, D), jnp.float32),
    )(outp, bias.reshape(1, D))
    return out


def kernel(in_feat, edge_list, W, att_src, att_dst, bias):
    return _run(in_feat, edge_list, W, att_src, att_dst, bias)
